# baseline (speedup 1.0000x reference)
import functools

import jax
import jax.numpy as jnp
from jax import lax
from jax.experimental import pallas as pl
from jax.experimental.pallas import tpu as pltpu
from jax.experimental.pallas import tpu_sc as plsc

_L = 16

_SC_PARAMS = pltpu.CompilerParams(
    needs_layout_passes=False, use_tc_tiling_on_sc=False)


def _tc_body(pts_ref, o_ref):
    P = _L
    pts = pts_ref[...]
    x1x, x2x = pts[:, 0:16], pts[:, 16:32]
    x1y, x2y = pts[:, 32:48], pts[:, 48:64]
    x1z, x2z = pts[:, 64:80], pts[:, 80:96]
    row = lax.broadcasted_iota(jnp.int32, (P, P * P), 0)
    col = lax.broadcasted_iota(jnp.int32, (P, P * P), 1)
    R1 = (row == col // P).astype(jnp.bfloat16)
    R2 = (row == col - (col // P) * P).astype(jnp.bfloat16)

    def fan(a, R):
        hi = a.astype(jnp.bfloat16)
        r1 = a - hi.astype(jnp.float32)
        mid = r1.astype(jnp.bfloat16)
        lo = (r1 - mid.astype(jnp.float32)).astype(jnp.bfloat16)
        a3 = jnp.concatenate([hi, mid, lo], axis=1)
        R3 = jnp.concatenate([R, R, R], axis=0)
        return lax.dot_general(
            a3, R3, (((1,), (0,)), ((), ())),
            preferred_element_type=jnp.float32)

    dx = fan(x1x, R1) - fan(x2x, R2)
    dy = fan(x1y, R1) - fan(x2y, R2)
    dz = fan(x1z, R1) - fan(x2z, R2)
    d = jnp.sqrt(dx * dx + dy * dy + dz * dz)
    m = jnp.min(d, axis=1, keepdims=True)
    io = lax.broadcasted_iota(jnp.int32, d.shape, 1)
    imin = jnp.min(jnp.where(d == m, io, P * P), axis=1, keepdims=True)
    i1 = imin // P
    i2 = imin - i1 * P
    io16 = lax.broadcasted_iota(jnp.int32, x1x.shape, 1)
    sel1 = io16 == i1
    sel2 = io16 == i2

    def pick(sel, a):
        return jnp.sum(jnp.where(sel, a, 0.0), axis=1, keepdims=True)

    v1x, v1y, v1z = pick(sel1, x1x), pick(sel1, x1y), pick(sel1, x1z)
    v2x, v2y, v2z = pick(sel2, x2x), pick(sel2, x2y), pick(sel2, x2z)
    dxv = v1x - v2x
    dyv = v1y - v2y
    dzv = v1z - v2z
    lend = jnp.sqrt(dxv * dxv + dyv * dyv + dzv * dzv)
    pos = lend > 0.0
    safe = jnp.where(pos, lend, 1.0)
    nx = jnp.where(pos, dxv / safe, dxv)
    ny = jnp.where(pos, dyv / safe, dyv)
    nz = jnp.where(pos, dzv / safe, dzv)
    feats = jnp.concatenate([
        v1x, v1y, v1z, v2x, v2y, v2z, nx, ny, nz, lend,
        nx * nx, nx * ny, nx * nz,
        ny * nx, ny * ny, ny * nz,
        nz * nx, nz * ny, nz * nz,
    ], axis=1)
    o_ref[...] = feats


def _gather_points(clusts_flat, planes, eidx_flat, NCL, V):
    P = _L
    E_pad = eidx_flat.shape[0] // 2
    info = plsc.get_sparse_core_info()
    NC, NS = info.num_cores, info.num_subcores
    NW = NC * NS
    HW = NW // 2

    CHUNK = 256
    W = (2 * E_pad) // NW
    assert W % CHUNK == 0

    mesh = plsc.VectorSubcoreMesh(core_axis_name="c", subcore_axis_name="s")

    NCHUNK = W // CHUNK
    BPC = CHUNK // 16

    @functools.partial(
        pl.kernel,
        mesh=mesh,
        out_type=jax.ShapeDtypeStruct((2 * E_pad * P,), jnp.int32),
        scratch_types=[
            pltpu.VMEM((NCL * P,), jnp.int32),
            pltpu.VMEM((W,), jnp.int32),
            pltpu.VMEM((BPC * 16 * P,), jnp.int32),
            pltpu.SemaphoreType.DMA,
        ],
        compiler_params=_SC_PARAMS,
    )
    def sc_resolve(clusts_h, eidx_h, vidx_h, clusts_v, ecs, vbuf, sem):
        w = lax.axis_index("s") * NC + lax.axis_index("c")
        base = w * W
        pltpu.sync_copy(clusts_h, clusts_v)
        pltpu.sync_copy(eidx_h.at[pl.ds(base, W)], ecs)
        iota = lax.iota(jnp.int32, _L)

        for t in range(NCHUNK):
            def batch(b, carry, t=t):
                j = t * BPC + b
                cvec = ecs[pl.ds(j * 16, 16)] * P
                for k in range(P):
                    vk = plsc.load_gather(clusts_v, [cvec + k])
                    plsc.store_scatter(vbuf, [b * 16 * P + iota * P + k], vk)
                pltpu.async_copy(
                    vbuf.at[pl.ds(b * 16 * P, 16 * P)],
                    vidx_h.at[pl.ds((base + j * 16) * P, 16 * P)], sem)
                return carry

            lax.fori_loop(0, BPC, batch, 0)
            for b in range(BPC):
                j = t * BPC + b
                pltpu.make_async_copy(
                    vbuf.at[pl.ds(b * 16 * P, 16 * P)],
                    vidx_h.at[pl.ds((base + j * 16) * P, 16 * P)], sem).wait()

    vidx = sc_resolve(clusts_flat, eidx_flat)

    @functools.partial(
        pl.kernel,
        mesh=mesh,
        out_type=jax.ShapeDtypeStruct((E_pad, 128), jnp.float32),
        scratch_types=[
            pltpu.VMEM((V,), jnp.float32),
            pltpu.VMEM((2, CHUNK * P), jnp.int32),
            pltpu.VMEM((BPC * 16, 16), jnp.float32),
            pltpu.SemaphoreType.DMA,
            pltpu.SemaphoreType.DMA,
        ],
        compiler_params=_SC_PARAMS,
    )
    def sc_planes(planes_h, vidx_h, out_h, plane_v, vbuf, och, semi, semo):
        w = lax.axis_index("s") * NC + lax.axis_index("c")
        s_ep = w // HW
        erow0 = (w % HW) * W
        vbase = w * W * P

        def chunk_src(t):
            return vidx_h.at[pl.ds(vbase + t * CHUNK * P, CHUNK * P)]

        for c in range(3):
            pltpu.sync_copy(planes_h.at[pl.ds(c * V, V)], plane_v)
            col0 = (2 * c) * 16 + s_ep * 16
            pltpu.async_copy(chunk_src(0), vbuf.at[0], semi)

            for t in range(NCHUNK):
                pltpu.make_async_copy(
                    chunk_src(t), vbuf.at[t % 2], semi).wait()
                if t + 1 < NCHUNK:
                    pltpu.async_copy(
                        chunk_src(t + 1), vbuf.at[(t + 1) % 2], semi)

                def batch(b, carry, t=t, col0=col0):
                    for l in range(16):
                        vi = vbuf[t % 2, pl.ds((b * 16 + l) * P, P)]
                        och[b * 16 + l, :] = plsc.load_gather(plane_v, [vi])
                    pltpu.async_copy(
                        och.at[pl.ds(b * 16, 16)],
                        out_h.at[pl.ds(erow0 + t * CHUNK + b * 16, 16),
                                 pl.ds(col0, 16)], semo)
                    return carry

                lax.fori_loop(0, BPC, batch, 0)
                for b in range(BPC):
                    pltpu.make_async_copy(
                        och.at[pl.ds(b * 16, 16)],
                        out_h.at[pl.ds(erow0 + t * CHUNK + b * 16, 16),
                                 pl.ds(col0, 16)], semo).wait()

    return sc_planes(planes, vidx)


def _tc_encode(pts):
    E_sub = pts.shape[0]
    Eb = min(2048, E_sub)
    return pl.pallas_call(
        _tc_body,
        grid=(E_sub // Eb,),
        in_specs=[pl.BlockSpec((Eb, 128), lambda i: (i, 0))],
        out_specs=pl.BlockSpec((Eb, 19), lambda i: (i, 0)),
        out_shape=jax.ShapeDtypeStruct((E_sub, 19), jnp.float32),
    )(pts)


def kernel(data, clusts, edge_index):
    E = edge_index.shape[1]
    NCL = clusts.shape[0]
    V = data.shape[0]
    NSPLIT = 2
    SUB = 16384

    eidx_pad = jnp.pad(edge_index, ((0, 0), (0, NSPLIT * SUB - E)))
    clusts_flat = clusts.reshape(-1)
    vox = data[:, :3]
    planes = jnp.concatenate([vox[:, 0], vox[:, 1], vox[:, 2]])

    feats = []
    for i in range(NSPLIT):
        eidx_i = eidx_pad[:, i * SUB:(i + 1) * SUB].reshape(-1)
        pts_i = _gather_points(clusts_flat, planes, eidx_i, NCL, V)
        feats.append(_tc_encode(pts_i))
    return jnp.concatenate(feats, axis=0)[:E]


# speedup vs baseline: 1.0750x; 1.0750x over previous
import functools

import jax
import jax.numpy as jnp
from jax import lax
from jax.experimental import pallas as pl
from jax.experimental.pallas import tpu as pltpu
from jax.experimental.pallas import tpu_sc as plsc

_L = 16

_SC_PARAMS = pltpu.CompilerParams(
    needs_layout_passes=False, use_tc_tiling_on_sc=False)


def _split3(a):
    hi = a.astype(jnp.bfloat16)
    r1 = a - hi.astype(jnp.float32)
    mid = r1.astype(jnp.bfloat16)
    lo = (r1 - mid.astype(jnp.float32)).astype(jnp.bfloat16)
    return hi, mid, lo


def _dot(a, b):
    return lax.dot_general(a, b, (((1,), (0,)), ((), ())),
                           preferred_element_type=jnp.float32)


def _tc_body(pts_ref, o_ref):
    P = _L
    pts = pts_ref[...]
    pts96 = pts[:, 0:96]
    phi, pmid, plo = _split3(pts96)

    row = lax.broadcasted_iota(jnp.int32, (P, P * P), 0)
    col = lax.broadcasted_iota(jnp.int32, (P, P * P), 1)
    R1 = jnp.concatenate([(row == col // P).astype(jnp.bfloat16)] * 3, axis=0)
    R2 = jnp.concatenate(
        [(row == col - (col // P) * P).astype(jnp.bfloat16)] * 3, axis=0)

    def fan(g, R):
        a3 = jnp.concatenate(
            [phi[:, g * 16:(g + 1) * 16], pmid[:, g * 16:(g + 1) * 16],
             plo[:, g * 16:(g + 1) * 16]], axis=1)
        return _dot(a3, R)

    dx = fan(0, R1) - fan(1, R2)
    dy = fan(2, R1) - fan(3, R2)
    dz = fan(4, R1) - fan(5, R2)
    d = jnp.sqrt(dx * dx + dy * dy + dz * dz)
    m = jnp.min(d, axis=1, keepdims=True)
    io = lax.broadcasted_iota(jnp.int32, d.shape, 1)
    imin = jnp.min(jnp.where(d == m, io, P * P), axis=1, keepdims=True)
    i1 = imin // P
    i2 = imin - i1 * P

    io96 = lax.broadcasted_iota(jnp.int32, pts96.shape, 1)
    targ = jnp.where((io96 >> 4) & 1 == 0, i1, i2)
    sel96 = (io96 & 15) == targ
    zb = jnp.zeros((), jnp.bfloat16)
    m3 = jnp.concatenate([jnp.where(sel96, phi, zb),
                          jnp.where(sel96, pmid, zb),
                          jnp.where(sel96, plo, zb)], axis=1)
    grow = lax.broadcasted_iota(jnp.int32, (96, 6), 0)
    gcol = lax.broadcasted_iota(jnp.int32, (96, 6), 1)
    G = (grow // 16 == (gcol % 3) * 2 + gcol // 3).astype(jnp.bfloat16)
    G3 = jnp.concatenate([G] * 3, axis=0)
    base6 = _dot(m3, G3)

    disp3 = base6[:, 0:3] - base6[:, 3:6]
    dxv, dyv, dzv = disp3[:, 0:1], disp3[:, 1:2], disp3[:, 2:3]
    lend = jnp.sqrt(dxv * dxv + dyv * dyv + dzv * dzv)
    pos = lend > 0.0
    safe = jnp.where(pos, lend, 1.0)
    n3 = jnp.where(pos, disp3 / safe, disp3)

    nhi, nmid, nlo = _split3(n3)
    n9 = jnp.concatenate([nhi, nmid, nlo], axis=1)
    prow = lax.broadcasted_iota(jnp.int32, (3, 9), 0)
    pcol = lax.broadcasted_iota(jnp.int32, (3, 9), 1)
    RP = jnp.concatenate([(prow == pcol // 3).astype(jnp.bfloat16)] * 3, 0)
    RT = jnp.concatenate(
        [(prow == pcol - (pcol // 3) * 3).astype(jnp.bfloat16)] * 3, 0)
    prod9 = _dot(n9, RP) * _dot(n9, RT)

    feats = jnp.concatenate([base6, n3, lend, prod9], axis=1)
    o_ref[...] = feats


def _gather_points(clusts_flat, planes, eidx_flat, NCL, V):
    P = _L
    E_pad = eidx_flat.shape[0] // 2
    info = plsc.get_sparse_core_info()
    NC, NS = info.num_cores, info.num_subcores
    NW = NC * NS
    HW = NW // 2

    CHUNK = 256
    W = (2 * E_pad) // NW
    assert W % CHUNK == 0

    mesh = plsc.VectorSubcoreMesh(core_axis_name="c", subcore_axis_name="s")

    NCHUNK = W // CHUNK
    BPC = CHUNK // 16

    @functools.partial(
        pl.kernel,
        mesh=mesh,
        out_type=jax.ShapeDtypeStruct((2 * E_pad * P,), jnp.int32),
        scratch_types=[
            pltpu.VMEM((NCL * P,), jnp.int32),
            pltpu.VMEM((W,), jnp.int32),
            pltpu.VMEM((BPC * 16 * P,), jnp.int32),
            pltpu.SemaphoreType.DMA,
        ],
        compiler_params=_SC_PARAMS,
    )
    def sc_resolve(clusts_h, eidx_h, vidx_h, clusts_v, ecs, vbuf, sem):
        w = lax.axis_index("s") * NC + lax.axis_index("c")
        base = w * W
        pltpu.sync_copy(clusts_h, clusts_v)
        pltpu.sync_copy(eidx_h.at[pl.ds(base, W)], ecs)
        iota = lax.iota(jnp.int32, _L)

        for t in range(NCHUNK):
            def batch(b, carry, t=t):
                j = t * BPC + b
                cvec = ecs[pl.ds(j * 16, 16)] * P
                for k in range(P):
                    vk = plsc.load_gather(clusts_v, [cvec + k])
                    plsc.store_scatter(vbuf, [b * 16 * P + iota * P + k], vk)
                pltpu.async_copy(
                    vbuf.at[pl.ds(b * 16 * P, 16 * P)],
                    vidx_h.at[pl.ds((base + j * 16) * P, 16 * P)], sem)
                return carry

            lax.fori_loop(0, BPC, batch, 0)
            for b in range(BPC):
                j = t * BPC + b
                pltpu.make_async_copy(
                    vbuf.at[pl.ds(b * 16 * P, 16 * P)],
                    vidx_h.at[pl.ds((base + j * 16) * P, 16 * P)], sem).wait()

    vidx = sc_resolve(clusts_flat, eidx_flat)

    @functools.partial(
        pl.kernel,
        mesh=mesh,
        out_type=jax.ShapeDtypeStruct((E_pad, 128), jnp.float32),
        scratch_types=[
            pltpu.VMEM((V,), jnp.float32),
            pltpu.VMEM((2, CHUNK * P), jnp.int32),
            pltpu.VMEM((BPC * 16, 16), jnp.float32),
            pltpu.SemaphoreType.DMA,
            pltpu.SemaphoreType.DMA,
        ],
        compiler_params=_SC_PARAMS,
    )
    def sc_planes(planes_h, vidx_h, out_h, plane_v, vbuf, och, semi, semo):
        w = lax.axis_index("s") * NC + lax.axis_index("c")
        s_ep = w // HW
        erow0 = (w % HW) * W
        vbase = w * W * P

        def chunk_src(t):
            return vidx_h.at[pl.ds(vbase + t * CHUNK * P, CHUNK * P)]

        for c in range(3):
            pltpu.sync_copy(planes_h.at[pl.ds(c * V, V)], plane_v)
            col0 = (2 * c) * 16 + s_ep * 16
            pltpu.async_copy(chunk_src(0), vbuf.at[0], semi)

            for t in range(NCHUNK):
                pltpu.make_async_copy(
                    chunk_src(t), vbuf.at[t % 2], semi).wait()
                if t + 1 < NCHUNK:
                    pltpu.async_copy(
                        chunk_src(t + 1), vbuf.at[(t + 1) % 2], semi)

                def batch(b, carry, t=t, col0=col0):
                    for l in range(16):
                        vi = vbuf[t % 2, pl.ds((b * 16 + l) * P, P)]
                        och[b * 16 + l, :] = plsc.load_gather(plane_v, [vi])
                    pltpu.async_copy(
                        och.at[pl.ds(b * 16, 16)],
                        out_h.at[pl.ds(erow0 + t * CHUNK + b * 16, 16),
                                 pl.ds(col0, 16)], semo)
                    return carry

                lax.fori_loop(0, BPC, batch, 0)
                for b in range(BPC):
                    pltpu.make_async_copy(
                        och.at[pl.ds(b * 16, 16)],
                        out_h.at[pl.ds(erow0 + t * CHUNK + b * 16, 16),
                                 pl.ds(col0, 16)], semo).wait()

    return sc_planes(planes, vidx)


def _tc_encode(pts):
    E_sub = pts.shape[0]
    Eb = min(2048, E_sub)
    return pl.pallas_call(
        _tc_body,
        grid=(E_sub // Eb,),
        in_specs=[pl.BlockSpec((Eb, 128), lambda i: (i, 0))],
        out_specs=pl.BlockSpec((Eb, 19), lambda i: (i, 0)),
        out_shape=jax.ShapeDtypeStruct((E_sub, 19), jnp.float32),
    )(pts)


def kernel(data, clusts, edge_index):
    E = edge_index.shape[1]
    NCL = clusts.shape[0]
    V = data.shape[0]
    NSPLIT = 2
    SUB = 16384

    eidx_pad = jnp.pad(edge_index, ((0, 0), (0, NSPLIT * SUB - E)))
    clusts_flat = clusts.reshape(-1)
    vox = data[:, :3]
    planes = jnp.concatenate([vox[:, 0], vox[:, 1], vox[:, 2]])

    feats = []
    for i in range(NSPLIT):
        eidx_i = eidx_pad[:, i * SUB:(i + 1) * SUB].reshape(-1)
        pts_i = _gather_points(clusts_flat, planes, eidx_i, NCL, V)
        feats.append(_tc_encode(pts_i))
    return jnp.concatenate(feats, axis=0)[:E]
